# baseline (speedup 1.0000x reference)
import jax
import jax.numpy as jnp
from jax import lax
from jax.experimental import pallas as pl
from jax.experimental.pallas import tpu as pltpu
from jax.experimental.pallas import tpu_sc as plsc

N = 10000
E = 320000
D = 128
HD = D // 2

NC = 2
NS = 16
EPT = E // NS
CH = 80
NCH = EPT // CH
NRCH = N // CH
RCPT = -(-NRCH // NS)
ZCH = 640
ZFULL = N // ZCH
ZREM = N - ZFULL * ZCH

BN = 400
NBLK = N // BN


def _mlp_body(h_ref, w1_ref, w2_ref, ac_ref, g1_ref, b1_ref, z2_ref, s_ref,
              t_ref):
    z1 = jnp.dot(h_ref[...], w1_ref[...], preferred_element_type=jnp.float32)
    z1 = jnp.maximum(z1 * g1_ref[...] + b1_ref[...], 0.0)
    z = jnp.dot(z1, w2_ref[...], preferred_element_type=jnp.float32)
    z2_ref[0] = z[:, :HD]
    z2_ref[1] = z[:, HD:]
    st = jnp.dot(z, ac_ref[...], preferred_element_type=jnp.float32)
    s_ref[...] = st[:, 0:1]
    t_ref[...] = st[:, 1:2]


def _sc_edge_body(z2_hbm, s_hbm, t_hbm, src_hbm, dst_hbm, gh_hbm, bh_hbm,
                  out_hbm,
                  s_v, t_v, src_v, dst_v, ee_v, rows_v, z1d_v, gb_v,
                  hout_sh, den_sh, gsem, ssem, dsem):
    c = lax.axis_index("c")
    sid = lax.axis_index("s")

    pltpu.sync_copy(s_hbm, s_v)
    pltpu.sync_copy(t_hbm, t_v)
    pltpu.sync_copy(gh_hbm.at[pl.ds(c * HD, HD)], gb_v.at[0])
    pltpu.sync_copy(bh_hbm.at[pl.ds(c * HD, HD)], gb_v.at[1])
    pltpu.sync_copy(src_hbm.at[sid], src_v)
    pltpu.sync_copy(dst_hbm.at[sid], dst_v)

    cofs = lax.broadcast(c * N, (16,))

    def off_body(j, carry):
        for k in range(CH // 16):
            sl = pl.ds(k * 16, 16)
            src_v[j, sl] = src_v[j, sl] + cofs
        return carry

    lax.fori_loop(0, NCH, off_body, 0)

    zv = jnp.zeros((16,), jnp.float32)

    def zrow_body(i, carry):
        for f in range(HD // 16):
            rows_v[0, i, pl.ds(f * 16, 16)] = zv
        return carry

    lax.fori_loop(0, CH, zrow_body, 0)
    for f in range(ZCH // 16):
        z1d_v[pl.ds(f * 16, 16)] = zv

    for cc in range(RCPT):
        rc = cc * NS + sid

        @pl.when(rc < NRCH)
        def _():
            pltpu.sync_copy(rows_v.at[0], hout_sh.at[pl.ds(rc * CH, CH)])

    @pl.when(sid < ZFULL)
    def _():
        pltpu.sync_copy(z1d_v, den_sh.at[pl.ds(sid * ZCH, ZCH)])

    @pl.when(sid == ZFULL)
    def _():
        pltpu.sync_copy(z1d_v.at[pl.ds(0, ZREM)], den_sh.at[pl.ds(ZFULL * ZCH, ZREM)])

    plsc.subcore_barrier()

    def drain(buf, sema):
        pltpu.make_async_copy(z2_hbm.at[pl.ds(0, CH)], rows_v.at[buf], sema).wait()

    def drain_ee(buf, sema):
        pltpu.make_async_copy(s_hbm.at[pl.ds(0, CH)], ee_v.at[buf], sema).wait()

    pltpu.async_copy(z2_hbm.at[src_v.at[0]], rows_v.at[0], gsem)

    lane_idx = [jnp.full((16,), i, jnp.int32) for i in range(16)]

    def sub_chunk(j, b):
        drain(b, gsem)

        @pl.when(j >= 1)
        def _():
            drain(1 - b, ssem)

        @pl.when(j >= 2)
        def _():
            drain_ee(b, dsem)

        @pl.when(j + 1 < NCH)
        def _():
            pltpu.async_copy(z2_hbm.at[src_v.at[j + 1]], rows_v.at[1 - b], gsem)

        for k in range(CH // 16):
            sv = src_v[j, pl.ds(k * 16, 16)]
            dv = dst_v[j, pl.ds(k * 16, 16)]
            sg = plsc.load_gather(s_v, [sv - cofs])
            tg = plsc.load_gather(t_v, [dv])
            eevec = jnp.exp(jnp.maximum(sg + tg, 0.0))
            ee_v[b, pl.ds(k * 16, 16)] = eevec
            for i in range(16):
                eb = eevec.at[lane_idx[i]].get(mode="promise_in_bounds")
                row = k * 16 + i
                for f in range(HD // 16):
                    rows_v[b, row, pl.ds(f * 16, 16)] = (
                        rows_v[b, row, pl.ds(f * 16, 16)] * eb)
        pltpu.async_copy(ee_v.at[b], den_sh.at[dst_v.at[j]], dsem, add=True)

        pltpu.async_copy(rows_v.at[b], hout_sh.at[dst_v.at[j]], ssem, add=True)

    def chunk_pair(jj, carry):
        sub_chunk(jj, 0)
        sub_chunk(jj + 1, 1)
        return carry

    lax.fori_loop(0, NCH // 2, lambda p, carry: chunk_pair(p * 2, carry), 0)
    drain(1, ssem)
    drain_ee(0, dsem)
    drain_ee(1, dsem)

    plsc.subcore_barrier()

    def wb_body(cc, carry):
        rc = cc * NS + sid

        @pl.when(rc < NRCH)
        def _():
            pltpu.sync_copy(hout_sh.at[pl.ds(rc * CH, CH)], rows_v.at[0])
            pltpu.sync_copy(den_sh.at[pl.ds(rc * CH, CH)], ee_v.at[0])
            ghv = [gb_v[0, pl.ds(f * 16, 16)] for f in range(HD // 16)]
            bhv = [gb_v[1, pl.ds(f * 16, 16)] for f in range(HD // 16)]
            for k in range(CH // 16):
                dvec = ee_v[0, pl.ds(k * 16, 16)]
                rec = 1.0 / jnp.where(dvec > 0.0, dvec, 1.0)
                for i in range(16):
                    rb = rec.at[lane_idx[i]].get(mode="promise_in_bounds")
                    row = k * 16 + i
                    for f in range(HD // 16):
                        rows_v[0, row, pl.ds(f * 16, 16)] = jnp.maximum(
                            rows_v[0, row, pl.ds(f * 16, 16)] * rb * ghv[f]
                            + bhv[f], 0.0)
            pltpu.sync_copy(rows_v.at[0],
                            out_hbm.at[pl.ds(rc * CH, CH), pl.ds(c * HD, HD)])
        return carry

    lax.fori_loop(0, RCPT, wb_body, 0)


def kernel(h, edge_index, W1, W2, A, g1, b1, gh, bh):
    acat = jnp.stack([A[:D], A[D:]], axis=1)

    z2, s1, t1 = pl.pallas_call(
        _mlp_body,
        grid=(NBLK,),
        in_specs=[
            pl.BlockSpec((BN, D), lambda i: (i, 0)),
            pl.BlockSpec((D, D), lambda i: (0, 0)),
            pl.BlockSpec((D, D), lambda i: (0, 0)),
            pl.BlockSpec((D, 2), lambda i: (0, 0)),
            pl.BlockSpec((1, D), lambda i: (0, 0)),
            pl.BlockSpec((1, D), lambda i: (0, 0)),
        ],
        out_specs=[
            pl.BlockSpec((2, BN, HD), lambda i: (0, i, 0)),
            pl.BlockSpec((BN, 1), lambda i: (i, 0)),
            pl.BlockSpec((BN, 1), lambda i: (i, 0)),
        ],
        out_shape=[
            jax.ShapeDtypeStruct((2, N, HD), jnp.float32),
            jax.ShapeDtypeStruct((N, 1), jnp.float32),
            jax.ShapeDtypeStruct((N, 1), jnp.float32),
        ],
    )(h, W1, W2, acat, g1.reshape(1, D), b1.reshape(1, D))

    s = s1.reshape(N)
    t = t1.reshape(N)
    z2f = z2.reshape(2 * N, HD)
    src3 = edge_index[0].reshape(NS, NCH, CH)
    dst3 = edge_index[1].reshape(NS, NCH, CH)

    mesh = plsc.VectorSubcoreMesh(core_axis_name="c", subcore_axis_name="s")
    out = pl.kernel(
        _sc_edge_body,
        mesh=mesh,
        compiler_params=pltpu.CompilerParams(
            needs_layout_passes=False, use_tc_tiling_on_sc=False,
            internal_scratch_in_bytes=128 * 1024),
        out_type=jax.ShapeDtypeStruct((N, D), jnp.float32),
        scratch_types=[
            pltpu.VMEM((N,), jnp.float32),
            pltpu.VMEM((N,), jnp.float32),
            pltpu.VMEM((NCH, CH), jnp.int32),
            pltpu.VMEM((NCH, CH), jnp.int32),
            pltpu.VMEM((2, CH), jnp.float32),
            pltpu.VMEM((2, CH, HD), jnp.float32),
            pltpu.VMEM((ZCH,), jnp.float32),
            pltpu.VMEM((2, HD), jnp.float32),
            pltpu.VMEM_SHARED((N, HD), jnp.float32),
            pltpu.VMEM_SHARED((N,), jnp.float32),
            pltpu.SemaphoreType.DMA,
            pltpu.SemaphoreType.DMA,
            pltpu.SemaphoreType.DMA,
        ],
    )(z2f, s, t, src3, dst3, gh, bh)

    return out


# speedup vs baseline: 1.0751x; 1.0751x over previous
import jax
import jax.numpy as jnp
from jax import lax
from jax.experimental import pallas as pl
from jax.experimental.pallas import tpu as pltpu
from jax.experimental.pallas import tpu_sc as plsc

N = 10000
E = 320000
D = 128
HD = D // 2

NC = 2
NS = 16
EPT = E // NS
CH = 80
NCH = EPT // CH
NRCH = N // CH
RCPT = -(-NRCH // NS)
ZCH = 640
ZFULL = N // ZCH
ZREM = N - ZFULL * ZCH

BN = 400
NBLK = N // BN


def _mlp_body(h_ref, w1_ref, w2_ref, ac_ref, g1_ref, b1_ref, z2_ref, s_ref,
              t_ref):
    z1 = jnp.dot(h_ref[...], w1_ref[...], preferred_element_type=jnp.float32)
    z1 = jnp.maximum(z1 * g1_ref[...] + b1_ref[...], 0.0)
    z = jnp.dot(z1, w2_ref[...], preferred_element_type=jnp.float32)
    z2_ref[0] = z[:, :HD].astype(jnp.bfloat16)
    z2_ref[1] = z[:, HD:].astype(jnp.bfloat16)
    st = jnp.dot(z, ac_ref[...], preferred_element_type=jnp.float32)
    s_ref[...] = st[:, 0:1]
    t_ref[...] = st[:, 1:2]


def _sc_edge_body(z2_hbm, s_hbm, t_hbm, src_hbm, dst_hbm, gh_hbm, bh_hbm,
                  out_hbm,
                  s_v, t_v, src_v, dst_v, ee_v, rows_bf, rows_v, z1d_v, gb_v,
                  hout_sh, den_sh, gsem, ssem, dsem):
    c = lax.axis_index("c")
    sid = lax.axis_index("s")

    pltpu.sync_copy(s_hbm, s_v)
    pltpu.sync_copy(t_hbm, t_v)
    pltpu.sync_copy(gh_hbm.at[pl.ds(c * HD, HD)], gb_v.at[0])
    pltpu.sync_copy(bh_hbm.at[pl.ds(c * HD, HD)], gb_v.at[1])
    pltpu.sync_copy(src_hbm.at[sid], src_v)
    pltpu.sync_copy(dst_hbm.at[sid], dst_v)

    cofs = lax.broadcast(c * N, (16,))

    def off_body(j, carry):
        for k in range(CH // 16):
            sl = pl.ds(k * 16, 16)
            src_v[j, sl] = src_v[j, sl] + cofs
        return carry

    lax.fori_loop(0, NCH, off_body, 0)

    zv = jnp.zeros((16,), jnp.float32)

    def zrow_body(i, carry):
        for f in range(HD // 16):
            rows_v[0, i, pl.ds(f * 16, 16)] = zv
        return carry

    lax.fori_loop(0, CH, zrow_body, 0)
    for f in range(ZCH // 16):
        z1d_v[pl.ds(f * 16, 16)] = zv

    for cc in range(RCPT):
        rc = cc * NS + sid

        @pl.when(rc < NRCH)
        def _():
            pltpu.sync_copy(rows_v.at[0], hout_sh.at[pl.ds(rc * CH, CH)])

    @pl.when(sid < ZFULL)
    def _():
        pltpu.sync_copy(z1d_v, den_sh.at[pl.ds(sid * ZCH, ZCH)])

    @pl.when(sid == ZFULL)
    def _():
        pltpu.sync_copy(z1d_v.at[pl.ds(0, ZREM)], den_sh.at[pl.ds(ZFULL * ZCH, ZREM)])

    plsc.subcore_barrier()

    def drain(buf, sema):
        pltpu.make_async_copy(z2_hbm.at[pl.ds(0, CH)], rows_bf.at[buf], sema).wait()

    def drain_s(buf, sema):
        pltpu.make_async_copy(out_hbm.at[pl.ds(0, CH), pl.ds(0, HD)],
                              rows_v.at[buf], sema).wait()

    def drain_ee(buf, sema):
        pltpu.make_async_copy(s_hbm.at[pl.ds(0, CH)], ee_v.at[buf], sema).wait()

    pltpu.async_copy(z2_hbm.at[src_v.at[0]], rows_bf.at[0], gsem)

    lane_idx = [jnp.full((16,), i, jnp.int32) for i in range(16)]

    def sub_chunk(j, b):
        drain(b, gsem)

        @pl.when(j >= 1)
        def _():
            drain_s(1 - b, ssem)

        @pl.when(j >= 2)
        def _():
            drain_ee(b, dsem)

        @pl.when(j + 1 < NCH)
        def _():
            pltpu.async_copy(z2_hbm.at[src_v.at[j + 1]], rows_bf.at[1 - b], gsem)

        for k in range(CH // 16):
            sv = src_v[j, pl.ds(k * 16, 16)]
            dv = dst_v[j, pl.ds(k * 16, 16)]
            sg = plsc.load_gather(s_v, [sv - cofs])
            tg = plsc.load_gather(t_v, [dv])
            eevec = jnp.exp(jnp.maximum(sg + tg, 0.0))
            ee_v[b, pl.ds(k * 16, 16)] = eevec
            for i in range(16):
                eb = eevec.at[lane_idx[i]].get(mode="promise_in_bounds")
                row = k * 16 + i
                for g in range(HD // 32):
                    bfv = rows_bf[b, row, pl.ds(g * 32, 32)]
                    lo, hi = plsc.unpack(bfv, format=plsc.PackFormat.INTERLEAVED)
                    rows_v[b, row, pl.ds(g * 32, 16)] = lo * eb
                    rows_v[b, row, pl.ds(g * 32 + 16, 16)] = hi * eb
        pltpu.async_copy(ee_v.at[b], den_sh.at[dst_v.at[j]], dsem, add=True)

        pltpu.async_copy(rows_v.at[b], hout_sh.at[dst_v.at[j]], ssem, add=True)

    def chunk_pair(jj, carry):
        sub_chunk(jj, 0)
        sub_chunk(jj + 1, 1)
        return carry

    lax.fori_loop(0, NCH // 2, lambda p, carry: chunk_pair(p * 2, carry), 0)
    drain_s(1, ssem)
    drain_ee(0, dsem)
    drain_ee(1, dsem)

    plsc.subcore_barrier()

    def wb_body(cc, carry):
        rc = cc * NS + sid

        @pl.when(rc < NRCH)
        def _():
            pltpu.sync_copy(hout_sh.at[pl.ds(rc * CH, CH)], rows_v.at[0])
            pltpu.sync_copy(den_sh.at[pl.ds(rc * CH, CH)], ee_v.at[0])
            ghv = [gb_v[0, pl.ds(f * 16, 16)] for f in range(HD // 16)]
            bhv = [gb_v[1, pl.ds(f * 16, 16)] for f in range(HD // 16)]
            for k in range(CH // 16):
                dvec = ee_v[0, pl.ds(k * 16, 16)]
                rec = 1.0 / jnp.where(dvec > 0.0, dvec, 1.0)
                for i in range(16):
                    rb = rec.at[lane_idx[i]].get(mode="promise_in_bounds")
                    row = k * 16 + i
                    for f in range(HD // 16):
                        rows_v[0, row, pl.ds(f * 16, 16)] = jnp.maximum(
                            rows_v[0, row, pl.ds(f * 16, 16)] * rb * ghv[f]
                            + bhv[f], 0.0)
            pltpu.sync_copy(rows_v.at[0],
                            out_hbm.at[pl.ds(rc * CH, CH), pl.ds(c * HD, HD)])
        return carry

    lax.fori_loop(0, RCPT, wb_body, 0)


_PERM = []
for _hh in range(2):
    for _g in range(2):
        _base = _hh * HD + _g * 32
        for _i in range(16):
            _PERM.append(_base + _i)
            _PERM.append(_base + 16 + _i)
_PERM = tuple(_PERM)


def kernel(h, edge_index, W1, W2, A, g1, b1, gh, bh):
    perm = jnp.array(_PERM, dtype=jnp.int32)
    W2p = W2[:, perm]
    acat = jnp.stack([A[:D], A[D:]], axis=1)[perm, :]

    z2, s1, t1 = pl.pallas_call(
        _mlp_body,
        grid=(NBLK,),
        in_specs=[
            pl.BlockSpec((BN, D), lambda i: (i, 0)),
            pl.BlockSpec((D, D), lambda i: (0, 0)),
            pl.BlockSpec((D, D), lambda i: (0, 0)),
            pl.BlockSpec((D, 2), lambda i: (0, 0)),
            pl.BlockSpec((1, D), lambda i: (0, 0)),
            pl.BlockSpec((1, D), lambda i: (0, 0)),
        ],
        out_specs=[
            pl.BlockSpec((2, BN, HD), lambda i: (0, i, 0)),
            pl.BlockSpec((BN, 1), lambda i: (i, 0)),
            pl.BlockSpec((BN, 1), lambda i: (i, 0)),
        ],
        out_shape=[
            jax.ShapeDtypeStruct((2, N, HD), jnp.bfloat16),
            jax.ShapeDtypeStruct((N, 1), jnp.float32),
            jax.ShapeDtypeStruct((N, 1), jnp.float32),
        ],
    )(h, W1, W2p, acat, g1.reshape(1, D), b1.reshape(1, D))

    s = s1.reshape(N)
    t = t1.reshape(N)
    z2f = z2.reshape(2 * N, HD)
    src3 = edge_index[0].reshape(NS, NCH, CH)
    dst3 = edge_index[1].reshape(NS, NCH, CH)

    mesh = plsc.VectorSubcoreMesh(core_axis_name="c", subcore_axis_name="s")
    out = pl.kernel(
        _sc_edge_body,
        mesh=mesh,
        compiler_params=pltpu.CompilerParams(
            needs_layout_passes=False, use_tc_tiling_on_sc=False,
            internal_scratch_in_bytes=128 * 1024),
        out_type=jax.ShapeDtypeStruct((N, D), jnp.float32),
        scratch_types=[
            pltpu.VMEM((N,), jnp.float32),
            pltpu.VMEM((N,), jnp.float32),
            pltpu.VMEM((NCH, CH), jnp.int32),
            pltpu.VMEM((NCH, CH), jnp.int32),
            pltpu.VMEM((2, CH), jnp.float32),
            pltpu.VMEM((2, CH, HD), jnp.bfloat16),
            pltpu.VMEM((2, CH, HD), jnp.float32),
            pltpu.VMEM((ZCH,), jnp.float32),
            pltpu.VMEM((2, HD), jnp.float32),
            pltpu.VMEM_SHARED((N, HD), jnp.float32),
            pltpu.VMEM_SHARED((N,), jnp.float32),
            pltpu.SemaphoreType.DMA,
            pltpu.SemaphoreType.DMA,
            pltpu.SemaphoreType.DMA,
        ],
    )(z2f, s, t, src3, dst3, gh, bh)

    return out


# speedup vs baseline: 1.0842x; 1.0085x over previous
import jax
import jax.numpy as jnp
from jax import lax
from jax.experimental import pallas as pl
from jax.experimental.pallas import tpu as pltpu
from jax.experimental.pallas import tpu_sc as plsc

N = 10000
E = 320000
D = 128
HD = D // 2

NC = 2
NS = 16
EPT = E // NS
CH = 80
NCH = EPT // CH
NRCH = N // CH
RCPT = -(-NRCH // NS)
ZCH = 640
ZFULL = N // ZCH
ZREM = N - ZFULL * ZCH

BN = 400
NBLK = N // BN


def _mlp_body(h_ref, w1_ref, w2_ref, ac_ref, g1_ref, b1_ref, z2_ref, s_ref,
              t_ref):
    z1 = jnp.dot(h_ref[...], w1_ref[...], preferred_element_type=jnp.float32)
    z1 = jnp.maximum(z1 * g1_ref[...] + b1_ref[...], 0.0)
    z = jnp.dot(z1, w2_ref[...], preferred_element_type=jnp.float32)
    z2_ref[0] = z[:, :HD].astype(jnp.bfloat16)
    z2_ref[1] = z[:, HD:].astype(jnp.bfloat16)
    st = jnp.dot(z, ac_ref[...], preferred_element_type=jnp.float32)
    s_ref[...] = st[:, 0:1]
    t_ref[...] = st[:, 1:2]


def _sc_edge_body(z2_hbm, s_hbm, t_hbm, src_hbm, dst_hbm, gh_hbm, bh_hbm,
                  out_hbm,
                  s_v, t_v, src_v, dst_v, ee_v, rows_bf, rows_v, z1d_v, gb_v,
                  hout_sh, den_sh, gsem, ssem, dsem):
    c = lax.axis_index("c")
    sid = lax.axis_index("s")

    pltpu.sync_copy(s_hbm, s_v)
    pltpu.sync_copy(t_hbm, t_v)
    pltpu.sync_copy(gh_hbm.at[pl.ds(c * HD, HD)], gb_v.at[0])
    pltpu.sync_copy(bh_hbm.at[pl.ds(c * HD, HD)], gb_v.at[1])
    pltpu.sync_copy(src_hbm.at[sid], src_v)
    pltpu.sync_copy(dst_hbm.at[sid], dst_v)

    cofs = lax.broadcast(c * N, (16,))

    def off_body(j, carry):
        for k in range(CH // 16):
            sl = pl.ds(k * 16, 16)
            src_v[j, sl] = src_v[j, sl] + cofs
        return carry

    lax.fori_loop(0, NCH, off_body, 0)

    zv = jnp.zeros((16,), jnp.float32)

    def zrow_body(i, carry):
        for f in range(HD // 16):
            rows_v[0, i, pl.ds(f * 16, 16)] = zv
        return carry

    lax.fori_loop(0, CH, zrow_body, 0)
    for f in range(ZCH // 16):
        z1d_v[pl.ds(f * 16, 16)] = zv

    for cc in range(RCPT):
        rc = cc * NS + sid

        @pl.when(rc < NRCH)
        def _():
            pltpu.sync_copy(rows_v.at[0], hout_sh.at[pl.ds(rc * CH, CH)])

    @pl.when(sid < ZFULL)
    def _():
        pltpu.sync_copy(z1d_v, den_sh.at[pl.ds(sid * ZCH, ZCH)])

    @pl.when(sid == ZFULL)
    def _():
        pltpu.sync_copy(z1d_v.at[pl.ds(0, ZREM)], den_sh.at[pl.ds(ZFULL * ZCH, ZREM)])

    plsc.subcore_barrier()

    def drain(buf, sema):
        pltpu.make_async_copy(z2_hbm.at[pl.ds(0, CH)], rows_bf.at[buf], sema).wait()

    def drain_s(buf, sema):
        pltpu.make_async_copy(out_hbm.at[pl.ds(0, CH), pl.ds(0, HD)],
                              rows_v.at[buf], sema).wait()

    def drain_ee(buf, sema):
        pltpu.make_async_copy(s_hbm.at[pl.ds(0, CH)], ee_v.at[buf], sema).wait()

    pltpu.async_copy(z2_hbm.at[src_v.at[0]], rows_bf.at[0], gsem)

    lane_idx = [jnp.full((16,), i, jnp.int32) for i in range(16)]

    def sub_chunk(j, b):
        drain(b, gsem)

        @pl.when(j >= 1)
        def _():
            drain_s(1 - b, ssem)

        @pl.when(j >= 2)
        def _():
            drain_ee(b, dsem)

        @pl.when(j + 1 < NCH)
        def _():
            pltpu.async_copy(z2_hbm.at[src_v.at[j + 1]], rows_bf.at[1 - b], gsem)

        for k in range(CH // 16):
            sv = src_v[j, pl.ds(k * 16, 16)]
            dv = dst_v[j, pl.ds(k * 16, 16)]
            sg = plsc.load_gather(s_v, [sv - cofs])
            tg = plsc.load_gather(t_v, [dv])
            eevec = jnp.exp(jnp.maximum(sg + tg, 0.0))
            ee_v[b, pl.ds(k * 16, 16)] = eevec
            for i in range(16):
                eb = eevec.at[lane_idx[i]].get(mode="promise_in_bounds")
                row = k * 16 + i
                for g in range(HD // 32):
                    bfv = rows_bf[b, row, pl.ds(g * 32, 32)]
                    lo, hi = plsc.unpack(bfv, format=plsc.PackFormat.INTERLEAVED)
                    rows_v[b, row, pl.ds(g * 32, 16)] = lo * eb
                    rows_v[b, row, pl.ds(g * 32 + 16, 16)] = hi * eb
        pltpu.async_copy(ee_v.at[b], den_sh.at[dst_v.at[j]], dsem, add=True)

        pltpu.async_copy(rows_v.at[b], hout_sh.at[dst_v.at[j]], ssem, add=True)

    def chunk_pair(jj, carry):
        sub_chunk(jj, 0)
        sub_chunk(jj + 1, 1)
        return carry

    lax.fori_loop(0, NCH // 2, lambda p, carry: chunk_pair(p * 2, carry), 0)
    drain_s(1, ssem)
    drain_ee(0, dsem)
    drain_ee(1, dsem)

    plsc.subcore_barrier()

    def wb_fetch(cc, b):
        rc = cc * NS + sid

        @pl.when(rc < NRCH)
        def _():
            pltpu.async_copy(hout_sh.at[pl.ds(rc * CH, CH)], rows_v.at[b], dsem)
            pltpu.async_copy(den_sh.at[pl.ds(rc * CH, CH)], ee_v.at[b], dsem)

    wb_fetch(0, 0)

    def wb_sub(cc, b):
        rc = cc * NS + sid

        @pl.when(rc < NRCH)
        def _():
            drain_s(b, dsem)
            drain_ee(b, dsem)
            wb_fetch(cc + 1, 1 - b)
            ghv = [gb_v[0, pl.ds(f * 16, 16)] for f in range(HD // 16)]
            bhv = [gb_v[1, pl.ds(f * 16, 16)] for f in range(HD // 16)]
            for k in range(CH // 16):
                dvec = ee_v[b, pl.ds(k * 16, 16)]
                rec = 1.0 / jnp.where(dvec > 0.0, dvec, 1.0)
                for i in range(16):
                    rb = rec.at[lane_idx[i]].get(mode="promise_in_bounds")
                    row = k * 16 + i
                    for f in range(HD // 16):
                        rows_v[b, row, pl.ds(f * 16, 16)] = jnp.maximum(
                            rows_v[b, row, pl.ds(f * 16, 16)] * rb * ghv[f]
                            + bhv[f], 0.0)
            pltpu.sync_copy(rows_v.at[b],
                            out_hbm.at[pl.ds(rc * CH, CH), pl.ds(c * HD, HD)])

    def wb_pair(p, carry):
        wb_sub(p * 2, 0)
        wb_sub(p * 2 + 1, 1)
        return carry

    lax.fori_loop(0, RCPT // 2, wb_pair, 0)


_PERM = []
for _hh in range(2):
    for _g in range(2):
        _base = _hh * HD + _g * 32
        for _i in range(16):
            _PERM.append(_base + _i)
            _PERM.append(_base + 16 + _i)
_PERM = tuple(_PERM)


def kernel(h, edge_index, W1, W2, A, g1, b1, gh, bh):
    perm = jnp.array(_PERM, dtype=jnp.int32)
    W2p = W2[:, perm]
    acat = jnp.stack([A[:D], A[D:]], axis=1)[perm, :]

    z2, s1, t1 = pl.pallas_call(
        _mlp_body,
        grid=(NBLK,),
        in_specs=[
            pl.BlockSpec((BN, D), lambda i: (i, 0)),
            pl.BlockSpec((D, D), lambda i: (0, 0)),
            pl.BlockSpec((D, D), lambda i: (0, 0)),
            pl.BlockSpec((D, 2), lambda i: (0, 0)),
            pl.BlockSpec((1, D), lambda i: (0, 0)),
            pl.BlockSpec((1, D), lambda i: (0, 0)),
        ],
        out_specs=[
            pl.BlockSpec((2, BN, HD), lambda i: (0, i, 0)),
            pl.BlockSpec((BN, 1), lambda i: (i, 0)),
            pl.BlockSpec((BN, 1), lambda i: (i, 0)),
        ],
        out_shape=[
            jax.ShapeDtypeStruct((2, N, HD), jnp.bfloat16),
            jax.ShapeDtypeStruct((N, 1), jnp.float32),
            jax.ShapeDtypeStruct((N, 1), jnp.float32),
        ],
    )(h, W1, W2p, acat, g1.reshape(1, D), b1.reshape(1, D))

    s = s1.reshape(N)
    t = t1.reshape(N)
    z2f = z2.reshape(2 * N, HD)
    src3 = edge_index[0].reshape(NS, NCH, CH)
    dst3 = edge_index[1].reshape(NS, NCH, CH)

    mesh = plsc.VectorSubcoreMesh(core_axis_name="c", subcore_axis_name="s")
    out = pl.kernel(
        _sc_edge_body,
        mesh=mesh,
        compiler_params=pltpu.CompilerParams(
            needs_layout_passes=False, use_tc_tiling_on_sc=False,
            internal_scratch_in_bytes=128 * 1024),
        out_type=jax.ShapeDtypeStruct((N, D), jnp.float32),
        scratch_types=[
            pltpu.VMEM((N,), jnp.float32),
            pltpu.VMEM((N,), jnp.float32),
            pltpu.VMEM((NCH, CH), jnp.int32),
            pltpu.VMEM((NCH, CH), jnp.int32),
            pltpu.VMEM((2, CH), jnp.float32),
            pltpu.VMEM((2, CH, HD), jnp.bfloat16),
            pltpu.VMEM((2, CH, HD), jnp.float32),
            pltpu.VMEM((ZCH,), jnp.float32),
            pltpu.VMEM((2, HD), jnp.float32),
            pltpu.VMEM_SHARED((N, HD), jnp.float32),
            pltpu.VMEM_SHARED((N,), jnp.float32),
            pltpu.SemaphoreType.DMA,
            pltpu.SemaphoreType.DMA,
            pltpu.SemaphoreType.DMA,
        ],
    )(z2f, s, t, src3, dst3, gh, bh)

    return out
